# Initial kernel scaffold; baseline (speedup 1.0000x reference)
import jax
import jax.numpy as jnp
from jax.experimental import pallas as pl


def kernel(x, edge_index, batch, enc_W, enc_b, W1, b1, W2, b2, out_W, out_b):
    raise NotImplementedError("write your pallas kernel here")


# speedup vs baseline: 12.2623x; 12.2623x over previous
import jax
import jax.numpy as jnp
from jax import lax
from jax.experimental import pallas as pl
from jax.experimental.pallas import tpu as pltpu
from jax.experimental.pallas import tpu_sc as plsc

N_NODES = 10000
N_EDGES = 320000
FEAT = 128
N_GRAPHS = 128
N_CLASS = 10

NC = 2
NS = 16
NW = NC * NS
EPW = N_EDGES // NW
CHUNK = 80
NSTEPS = EPW // CHUNK
N_PAD = 10240
ROWS_PT = N_PAD // NS

_SC_MESH = plsc.VectorSubcoreMesh(
    core_axis_name="c", subcore_axis_name="s", num_cores=NC, num_subcores=NS)

ROWBLK = 1000
NBLK = N_NODES // ROWBLK


def _sc_degree_body(dst_hbm, zeros_hbm, ones_hbm, out_hbm, dst_v, ones_v,
                    acc_sh):
    c = lax.axis_index("c")
    s = lax.axis_index("s")
    wid = s * NC + c
    pltpu.sync_copy(zeros_hbm.at[pl.ds(s * ROWS_PT, ROWS_PT)],
                    acc_sh.at[pl.ds(s * ROWS_PT, ROWS_PT)])
    pltpu.sync_copy(ones_hbm, ones_v)
    plsc.subcore_barrier()
    base0 = wid * EPW

    def step(i, carry):
        base = pl.multiple_of(base0 + i * CHUNK, 8)
        pltpu.sync_copy(dst_hbm.at[pl.ds(base, CHUNK)], dst_v)
        pltpu.sync_copy(ones_v, acc_sh.at[dst_v], add=True)
        return carry

    lax.fori_loop(0, NSTEPS, step, 0)
    plsc.subcore_barrier()
    pltpu.sync_copy(acc_sh.at[pl.ds(s * ROWS_PT, ROWS_PT)],
                    out_hbm.at[c, pl.ds(s * ROWS_PT, ROWS_PT)])


_sc_degree = pl.kernel(
    _sc_degree_body,
    out_type=jax.ShapeDtypeStruct((NC, N_PAD, FEAT), jnp.float32),
    mesh=_SC_MESH,
    scratch_types=[
        pltpu.VMEM((CHUNK,), jnp.int32),
        pltpu.VMEM((CHUNK, FEAT), jnp.float32),
        pltpu.VMEM_SHARED((N_PAD, FEAT), jnp.float32),
    ],
)


def _sc_scatter_body(g_hbm, src_hbm, dst_hbm, zeros_hbm, out_hbm,
                     src_v, dst_v, rows_v, sem, acc_sh):
    c = lax.axis_index("c")
    s = lax.axis_index("s")
    wid = s * NC + c
    pltpu.sync_copy(zeros_hbm.at[pl.ds(s * ROWS_PT, ROWS_PT)],
                    acc_sh.at[pl.ds(s * ROWS_PT, ROWS_PT)])
    plsc.subcore_barrier()
    base0 = wid * EPW

    def step(i, carry):
        base = pl.multiple_of(base0 + i * CHUNK, 8)
        pltpu.sync_copy(src_hbm.at[pl.ds(base, CHUNK)], src_v)
        pltpu.sync_copy(dst_hbm.at[pl.ds(base, CHUNK)], dst_v)
        pltpu.async_copy(g_hbm.at[src_v], rows_v, sem).wait()
        pltpu.sync_copy(rows_v, acc_sh.at[dst_v], add=True)
        return carry

    lax.fori_loop(0, NSTEPS, step, 0)
    plsc.subcore_barrier()
    pltpu.sync_copy(acc_sh.at[pl.ds(s * ROWS_PT, ROWS_PT)],
                    out_hbm.at[c, pl.ds(s * ROWS_PT, ROWS_PT)])


_sc_scatter = pl.kernel(
    _sc_scatter_body,
    out_type=jax.ShapeDtypeStruct((NC, N_PAD, FEAT), jnp.float32),
    mesh=_SC_MESH,
    scratch_types=[
        pltpu.VMEM((CHUNK,), jnp.int32),
        pltpu.VMEM((CHUNK,), jnp.int32),
        pltpu.VMEM((CHUNK, FEAT), jnp.float32),
        pltpu.SemaphoreType.DMA,
        pltpu.VMEM_SHARED((N_PAD, FEAT), jnp.float32),
    ],
)


def _tc_enc_body(x_ref, encW_ref, encb_ref, W1_ref, m1_ref):
    h0 = jnp.dot(x_ref[...], encW_ref[...],
                 preferred_element_type=jnp.float32) + encb_ref[...]
    m1_ref[...] = jnp.dot(h0, W1_ref[...], preferred_element_type=jnp.float32)


def _tc_enc(x, enc_W, enc_b2, W1):
    return pl.pallas_call(
        _tc_enc_body,
        grid=(NBLK,),
        in_specs=[
            pl.BlockSpec((ROWBLK, FEAT), lambda i: (i, 0)),
            pl.BlockSpec((FEAT, FEAT), lambda i: (0, 0)),
            pl.BlockSpec((1, FEAT), lambda i: (0, 0)),
            pl.BlockSpec((FEAT, FEAT), lambda i: (0, 0)),
        ],
        out_specs=pl.BlockSpec((ROWBLK, FEAT), lambda i: (i, 0)),
        out_shape=jax.ShapeDtypeStruct((N_NODES, FEAT), jnp.float32),
    )(x, enc_W, enc_b2, W1)


def _tc_g1_body(degp_ref, m1_ref, g1_ref, dinv_ref):
    deg = degp_ref[0, :, 0:1] + degp_ref[1, :, 0:1] + 1.0
    dinv = lax.rsqrt(deg)
    g1_ref[...] = dinv * m1_ref[...]
    dinv_ref[...] = jnp.broadcast_to(dinv, (ROWBLK, 8))


def _tc_g1(deg_parts, m1):
    return pl.pallas_call(
        _tc_g1_body,
        grid=(NBLK,),
        in_specs=[
            pl.BlockSpec((NC, ROWBLK, FEAT), lambda i: (0, i, 0)),
            pl.BlockSpec((ROWBLK, FEAT), lambda i: (i, 0)),
        ],
        out_specs=[
            pl.BlockSpec((ROWBLK, FEAT), lambda i: (i, 0)),
            pl.BlockSpec((ROWBLK, 8), lambda i: (i, 0)),
        ],
        out_shape=[
            jax.ShapeDtypeStruct((N_NODES, FEAT), jnp.float32),
            jax.ShapeDtypeStruct((N_NODES, 8), jnp.float32),
        ],
    )(deg_parts, m1)


def _tc_mid_body(acc_ref, g_ref, dinv_ref, b_ref, W_ref, out_ref):
    dinv = dinv_ref[:, 0:1]
    h = jnp.maximum(
        dinv * (acc_ref[0] + acc_ref[1] + g_ref[...]) + b_ref[...], 0.0)
    out_ref[...] = dinv * jnp.dot(h, W_ref[...],
                                  preferred_element_type=jnp.float32)


def _tc_mid(acc, g, dinv8, b2d, W):
    return pl.pallas_call(
        _tc_mid_body,
        grid=(NBLK,),
        in_specs=[
            pl.BlockSpec((NC, ROWBLK, FEAT), lambda i: (0, i, 0)),
            pl.BlockSpec((ROWBLK, FEAT), lambda i: (i, 0)),
            pl.BlockSpec((ROWBLK, 8), lambda i: (i, 0)),
            pl.BlockSpec((1, FEAT), lambda i: (0, 0)),
            pl.BlockSpec((FEAT, FEAT), lambda i: (0, 0)),
        ],
        out_specs=pl.BlockSpec((ROWBLK, FEAT), lambda i: (i, 0)),
        out_shape=jax.ShapeDtypeStruct((N_NODES, FEAT), jnp.float32),
    )(acc, g, dinv8, b2d, W)


def _tc_final_body(acc_ref, g_ref, dinv_ref, b_ref, batch_ref, outW_ref,
                   outb_ref, out_ref, sums_ref, counts_ref):
    i = pl.program_id(0)
    dinv = dinv_ref[:, 0:1]
    h = jnp.maximum(
        dinv * (acc_ref[0] + acc_ref[1] + g_ref[...]) + b_ref[...], 0.0)
    onehot = (batch_ref[:, 0:1] ==
              lax.broadcasted_iota(jnp.int32, (ROWBLK, N_GRAPHS), 1)
              ).astype(jnp.float32)
    psum = lax.dot_general(onehot, h, (((0,), (0,)), ((), ())),
                           preferred_element_type=jnp.float32)
    pcnt = lax.dot_general(onehot, jnp.ones_like(h), (((0,), (0,)), ((), ())),
                           preferred_element_type=jnp.float32)

    @pl.when(i == 0)
    def _():
        sums_ref[...] = jnp.zeros_like(sums_ref)
        counts_ref[...] = jnp.zeros_like(counts_ref)

    sums_ref[...] += psum
    counts_ref[...] += pcnt

    @pl.when(i == pl.num_programs(0) - 1)
    def _():
        pooled = sums_ref[...] / jnp.maximum(counts_ref[...], 1.0)
        out_ref[...] = jnp.dot(pooled, outW_ref[...],
                               preferred_element_type=jnp.float32) + outb_ref[...]


def _tc_final(acc, g, dinv8, b2d, batch8, out_W, out_b2):
    return pl.pallas_call(
        _tc_final_body,
        grid=(NBLK,),
        in_specs=[
            pl.BlockSpec((NC, ROWBLK, FEAT), lambda i: (0, i, 0)),
            pl.BlockSpec((ROWBLK, FEAT), lambda i: (i, 0)),
            pl.BlockSpec((ROWBLK, 8), lambda i: (i, 0)),
            pl.BlockSpec((1, FEAT), lambda i: (0, 0)),
            pl.BlockSpec((ROWBLK, 8), lambda i: (i, 0)),
            pl.BlockSpec((FEAT, N_CLASS), lambda i: (0, 0)),
            pl.BlockSpec((1, N_CLASS), lambda i: (0, 0)),
        ],
        out_specs=pl.BlockSpec((N_GRAPHS, N_CLASS), lambda i: (0, 0)),
        out_shape=jax.ShapeDtypeStruct((N_GRAPHS, N_CLASS), jnp.float32),
        scratch_shapes=[
            pltpu.VMEM((N_GRAPHS, FEAT), jnp.float32),
            pltpu.VMEM((N_GRAPHS, FEAT), jnp.float32),
        ],
    )(acc, g, dinv8, b2d, batch8, out_W, out_b2)


def kernel(x, edge_index, batch, enc_W, enc_b, W1, b1, W2, b2, out_W, out_b):
    src = edge_index[0].astype(jnp.int32)
    dst = edge_index[1].astype(jnp.int32)
    zeros128 = jnp.zeros((N_PAD, FEAT), jnp.float32)
    batch8 = jnp.broadcast_to(batch.astype(jnp.int32)[:, None], (N_NODES, 8))
    enc_b2 = enc_b.reshape(1, FEAT)
    b1_2 = b1.reshape(1, FEAT)
    b2_2 = b2.reshape(1, FEAT)
    out_b2 = out_b.reshape(1, N_CLASS)

    ones128 = jnp.ones((CHUNK, FEAT), jnp.float32)
    deg_parts = _sc_degree(dst, zeros128, ones128)
    m1 = _tc_enc(x, enc_W, enc_b2, W1)
    g1, dinv8 = _tc_g1(deg_parts, m1)
    acc1 = _sc_scatter(g1, src, dst, zeros128)
    g2 = _tc_mid(acc1, g1, dinv8, b1_2, W2)
    acc2 = _sc_scatter(g2, src, dst, zeros128)
    return _tc_final(acc2, g2, dinv8, b2_2, batch8, out_W, out_b2)


# speedup vs baseline: 19.4462x; 1.5858x over previous
import jax
import jax.numpy as jnp
from jax import lax
from jax.experimental import pallas as pl
from jax.experimental.pallas import tpu as pltpu
from jax.experimental.pallas import tpu_sc as plsc

N_NODES = 10000
N_EDGES = 320000
FEAT = 128
N_GRAPHS = 128
N_CLASS = 10

NC = 2
NS = 16
NW = NC * NS
EPW = N_EDGES // NW
CHUNK = 80
NSTEPS = EPW // CHUNK
N_PAD = 10240
ROWS_PT = N_PAD // NS

_SC_MESH = plsc.VectorSubcoreMesh(
    core_axis_name="c", subcore_axis_name="s", num_cores=NC, num_subcores=NS)

ROWBLK = 1000
NBLK = N_NODES // ROWBLK


def _sc_degree_body(dst_hbm, zeros_hbm, ones_hbm, out_hbm, dst_v, ones_v,
                    ssem0, ssem1, acc_sh):
    c = lax.axis_index("c")
    s = lax.axis_index("s")
    wid = s * NC + c
    pltpu.sync_copy(zeros_hbm.at[pl.ds(s * ROWS_PT, ROWS_PT)],
                    acc_sh.at[pl.ds(s * ROWS_PT, ROWS_PT)])
    pltpu.sync_copy(ones_hbm, ones_v)
    plsc.subcore_barrier()
    base0 = wid * EPW
    ssems = (ssem0, ssem1)

    def load_idx(ci, b):
        base = pl.multiple_of(base0 + ci * CHUNK, 8)
        pltpu.sync_copy(dst_hbm.at[pl.ds(base, CHUNK)], dst_v.at[b])

    def scat_start(b):
        pltpu.async_copy(ones_v, acc_sh.at[dst_v.at[b]], ssems[b], add=True)

    def scat_wait(b):
        pltpu.make_async_copy(ones_v, acc_sh.at[dst_v.at[b]], ssems[b]).wait()

    load_idx(0, 0)
    scat_start(0)
    load_idx(1, 1)
    scat_start(1)
    scat_wait(0)
    load_idx(2, 0)
    scat_start(0)

    def step(t, carry):
        for k in range(2):
            ci = 3 + 2 * t + k
            b = (1 + k) % 2
            scat_wait(b)
            load_idx(ci, b)
            scat_start(b)
        return carry

    lax.fori_loop(0, (NSTEPS - 3) // 2, step, 0)
    scat_wait(1)
    scat_wait(0)
    plsc.subcore_barrier()
    pltpu.sync_copy(acc_sh.at[pl.ds(s * ROWS_PT, ROWS_PT)],
                    out_hbm.at[c, pl.ds(s * ROWS_PT, ROWS_PT)])


_sc_degree = pl.kernel(
    _sc_degree_body,
    out_type=jax.ShapeDtypeStruct((NC, N_PAD, FEAT), jnp.float32),
    mesh=_SC_MESH,
    scratch_types=[
        pltpu.VMEM((2, CHUNK), jnp.int32),
        pltpu.VMEM((CHUNK, FEAT), jnp.float32),
        pltpu.SemaphoreType.DMA,
        pltpu.SemaphoreType.DMA,
        pltpu.VMEM_SHARED((N_PAD, FEAT), jnp.float32),
    ],
)


def _sc_scatter_body(g_hbm, src_hbm, dst_hbm, zeros_hbm, out_hbm,
                     src_v, dst_v, rows_v, gsem0, gsem1, ssem0, ssem1,
                     acc_sh):
    c = lax.axis_index("c")
    s = lax.axis_index("s")
    wid = s * NC + c
    pltpu.sync_copy(zeros_hbm.at[pl.ds(s * ROWS_PT, ROWS_PT)],
                    acc_sh.at[pl.ds(s * ROWS_PT, ROWS_PT)])
    plsc.subcore_barrier()
    base0 = wid * EPW
    gsems = (gsem0, gsem1)
    ssems = (ssem0, ssem1)

    def load_gather(ci, b):
        base = pl.multiple_of(base0 + ci * CHUNK, 8)
        pltpu.sync_copy(src_hbm.at[pl.ds(base, CHUNK)], src_v.at[b])
        pltpu.sync_copy(dst_hbm.at[pl.ds(base, CHUNK)], dst_v.at[b])
        pltpu.async_copy(g_hbm.at[src_v.at[b]], rows_v.at[b], gsems[b])

    def gather_wait(b):
        pltpu.make_async_copy(g_hbm.at[src_v.at[b]], rows_v.at[b],
                              gsems[b]).wait()

    def scat_start(b):
        pltpu.async_copy(rows_v.at[b], acc_sh.at[dst_v.at[b]], ssems[b],
                         add=True)

    def scat_wait(b):
        pltpu.make_async_copy(rows_v.at[b], acc_sh.at[dst_v.at[b]],
                              ssems[b]).wait()

    load_gather(0, 0)
    load_gather(1, 1)
    gather_wait(0)
    scat_start(0)
    scat_wait(0)
    load_gather(2, 0)
    gather_wait(1)
    scat_start(1)

    def step(t, carry):
        for k in range(2):
            ci = 3 + 2 * t + k
            b = (1 + k) % 2
            scat_wait(b)
            load_gather(ci, b)
            gather_wait((b + 1) % 2)
            scat_start((b + 1) % 2)
        return carry

    lax.fori_loop(0, (NSTEPS - 3) // 2, step, 0)
    gather_wait(0)
    scat_start(0)
    scat_wait(1)
    scat_wait(0)
    plsc.subcore_barrier()
    pltpu.sync_copy(acc_sh.at[pl.ds(s * ROWS_PT, ROWS_PT)],
                    out_hbm.at[c, pl.ds(s * ROWS_PT, ROWS_PT)])


_sc_scatter = pl.kernel(
    _sc_scatter_body,
    out_type=jax.ShapeDtypeStruct((NC, N_PAD, FEAT), jnp.float32),
    mesh=_SC_MESH,
    scratch_types=[
        pltpu.VMEM((2, CHUNK), jnp.int32),
        pltpu.VMEM((2, CHUNK), jnp.int32),
        pltpu.VMEM((2, CHUNK, FEAT), jnp.float32),
        pltpu.SemaphoreType.DMA,
        pltpu.SemaphoreType.DMA,
        pltpu.SemaphoreType.DMA,
        pltpu.SemaphoreType.DMA,
        pltpu.VMEM_SHARED((N_PAD, FEAT), jnp.float32),
    ],
)


def _tc_enc_body(x_ref, encW_ref, encb_ref, W1_ref, m1_ref):
    h0 = jnp.dot(x_ref[...], encW_ref[...],
                 preferred_element_type=jnp.float32) + encb_ref[...]
    m1_ref[...] = jnp.dot(h0, W1_ref[...], preferred_element_type=jnp.float32)


def _tc_enc(x, enc_W, enc_b2, W1):
    return pl.pallas_call(
        _tc_enc_body,
        grid=(NBLK,),
        in_specs=[
            pl.BlockSpec((ROWBLK, FEAT), lambda i: (i, 0)),
            pl.BlockSpec((FEAT, FEAT), lambda i: (0, 0)),
            pl.BlockSpec((1, FEAT), lambda i: (0, 0)),
            pl.BlockSpec((FEAT, FEAT), lambda i: (0, 0)),
        ],
        out_specs=pl.BlockSpec((ROWBLK, FEAT), lambda i: (i, 0)),
        out_shape=jax.ShapeDtypeStruct((N_NODES, FEAT), jnp.float32),
    )(x, enc_W, enc_b2, W1)


def _tc_g1_body(degp_ref, m1_ref, g1_ref, dinv_ref):
    deg = degp_ref[0, :, 0:1] + degp_ref[1, :, 0:1] + 1.0
    dinv = lax.rsqrt(deg)
    g1_ref[...] = dinv * m1_ref[...]
    dinv_ref[...] = jnp.broadcast_to(dinv, (ROWBLK, 8))


def _tc_g1(deg_parts, m1):
    return pl.pallas_call(
        _tc_g1_body,
        grid=(NBLK,),
        in_specs=[
            pl.BlockSpec((NC, ROWBLK, FEAT), lambda i: (0, i, 0)),
            pl.BlockSpec((ROWBLK, FEAT), lambda i: (i, 0)),
        ],
        out_specs=[
            pl.BlockSpec((ROWBLK, FEAT), lambda i: (i, 0)),
            pl.BlockSpec((ROWBLK, 8), lambda i: (i, 0)),
        ],
        out_shape=[
            jax.ShapeDtypeStruct((N_NODES, FEAT), jnp.float32),
            jax.ShapeDtypeStruct((N_NODES, 8), jnp.float32),
        ],
    )(deg_parts, m1)


def _tc_mid_body(acc_ref, g_ref, dinv_ref, b_ref, W_ref, out_ref):
    dinv = dinv_ref[:, 0:1]
    h = jnp.maximum(
        dinv * (acc_ref[0] + acc_ref[1] + g_ref[...]) + b_ref[...], 0.0)
    out_ref[...] = dinv * jnp.dot(h, W_ref[...],
                                  preferred_element_type=jnp.float32)


def _tc_mid(acc, g, dinv8, b2d, W):
    return pl.pallas_call(
        _tc_mid_body,
        grid=(NBLK,),
        in_specs=[
            pl.BlockSpec((NC, ROWBLK, FEAT), lambda i: (0, i, 0)),
            pl.BlockSpec((ROWBLK, FEAT), lambda i: (i, 0)),
            pl.BlockSpec((ROWBLK, 8), lambda i: (i, 0)),
            pl.BlockSpec((1, FEAT), lambda i: (0, 0)),
            pl.BlockSpec((FEAT, FEAT), lambda i: (0, 0)),
        ],
        out_specs=pl.BlockSpec((ROWBLK, FEAT), lambda i: (i, 0)),
        out_shape=jax.ShapeDtypeStruct((N_NODES, FEAT), jnp.float32),
    )(acc, g, dinv8, b2d, W)


def _tc_final_body(acc_ref, g_ref, dinv_ref, b_ref, batch_ref, outW_ref,
                   outb_ref, out_ref, sums_ref, counts_ref):
    i = pl.program_id(0)
    dinv = dinv_ref[:, 0:1]
    h = jnp.maximum(
        dinv * (acc_ref[0] + acc_ref[1] + g_ref[...]) + b_ref[...], 0.0)
    onehot = (batch_ref[:, 0:1] ==
              lax.broadcasted_iota(jnp.int32, (ROWBLK, N_GRAPHS), 1)
              ).astype(jnp.float32)
    psum = lax.dot_general(onehot, h, (((0,), (0,)), ((), ())),
                           preferred_element_type=jnp.float32)
    pcnt = lax.dot_general(onehot, jnp.ones_like(h), (((0,), (0,)), ((), ())),
                           preferred_element_type=jnp.float32)

    @pl.when(i == 0)
    def _():
        sums_ref[...] = jnp.zeros_like(sums_ref)
        counts_ref[...] = jnp.zeros_like(counts_ref)

    sums_ref[...] += psum
    counts_ref[...] += pcnt

    @pl.when(i == pl.num_programs(0) - 1)
    def _():
        pooled = sums_ref[...] / jnp.maximum(counts_ref[...], 1.0)
        out_ref[...] = jnp.dot(pooled, outW_ref[...],
                               preferred_element_type=jnp.float32) + outb_ref[...]


def _tc_final(acc, g, dinv8, b2d, batch8, out_W, out_b2):
    return pl.pallas_call(
        _tc_final_body,
        grid=(NBLK,),
        in_specs=[
            pl.BlockSpec((NC, ROWBLK, FEAT), lambda i: (0, i, 0)),
            pl.BlockSpec((ROWBLK, FEAT), lambda i: (i, 0)),
            pl.BlockSpec((ROWBLK, 8), lambda i: (i, 0)),
            pl.BlockSpec((1, FEAT), lambda i: (0, 0)),
            pl.BlockSpec((ROWBLK, 8), lambda i: (i, 0)),
            pl.BlockSpec((FEAT, N_CLASS), lambda i: (0, 0)),
            pl.BlockSpec((1, N_CLASS), lambda i: (0, 0)),
        ],
        out_specs=pl.BlockSpec((N_GRAPHS, N_CLASS), lambda i: (0, 0)),
        out_shape=jax.ShapeDtypeStruct((N_GRAPHS, N_CLASS), jnp.float32),
        scratch_shapes=[
            pltpu.VMEM((N_GRAPHS, FEAT), jnp.float32),
            pltpu.VMEM((N_GRAPHS, FEAT), jnp.float32),
        ],
    )(acc, g, dinv8, b2d, batch8, out_W, out_b2)


def kernel(x, edge_index, batch, enc_W, enc_b, W1, b1, W2, b2, out_W, out_b):
    src = edge_index[0].astype(jnp.int32)
    dst = edge_index[1].astype(jnp.int32)
    zeros128 = jnp.zeros((N_PAD, FEAT), jnp.float32)
    batch8 = jnp.broadcast_to(batch.astype(jnp.int32)[:, None], (N_NODES, 8))
    enc_b2 = enc_b.reshape(1, FEAT)
    b1_2 = b1.reshape(1, FEAT)
    b2_2 = b2.reshape(1, FEAT)
    out_b2 = out_b.reshape(1, N_CLASS)

    ones128 = jnp.ones((CHUNK, FEAT), jnp.float32)
    deg_parts = _sc_degree(dst, zeros128, ones128)
    m1 = _tc_enc(x, enc_W, enc_b2, W1)
    g1, dinv8 = _tc_g1(deg_parts, m1)
    acc1 = _sc_scatter(g1, src, dst, zeros128)
    g2 = _tc_mid(acc1, g1, dinv8, b1_2, W2)
    acc2 = _sc_scatter(g2, src, dst, zeros128)
    return _tc_final(acc2, g2, dinv8, b2_2, batch8, out_W, out_b2)


# speedup vs baseline: 27.5238x; 1.4154x over previous
import jax
import jax.numpy as jnp
from jax import lax
from jax.experimental import pallas as pl
from jax.experimental.pallas import tpu as pltpu
from jax.experimental.pallas import tpu_sc as plsc

N_NODES = 10000
N_EDGES = 320000
FEAT = 128
N_GRAPHS = 128
N_CLASS = 10

NC = 2
NS = 16
NW = NC * NS
CHUNK = 128
NSTEPS = 79
E_PAD = NW * NSTEPS * CHUNK
N_PAD = 10240
ROWS_PT = N_PAD // NS

_SC_MESH = plsc.VectorSubcoreMesh(
    core_axis_name="c", subcore_axis_name="s", num_cores=NC, num_subcores=NS)

ROWBLK = 1000
NBLK = N_NODES // ROWBLK


def _sc_degree_body(dstw_hbm, zeros_hbm, ones_hbm, out_hbm, idx_d, ones_v,
                    ssem0, ssem1, ssem2, ssem3, acc_sh):
    c = lax.axis_index("c")
    s = lax.axis_index("s")
    wid = s * NC + c
    pltpu.sync_copy(zeros_hbm.at[pl.ds(s * ROWS_PT, ROWS_PT)],
                    acc_sh.at[pl.ds(s * ROWS_PT, ROWS_PT)])
    pltpu.sync_copy(dstw_hbm.at[wid], idx_d)
    pltpu.sync_copy(ones_hbm, ones_v)
    plsc.subcore_barrier()
    ssems = (ssem0, ssem1, ssem2, ssem3)

    def scat_start(ci, b):
        pltpu.async_copy(ones_v, acc_sh.at[idx_d.at[ci]], ssems[b], add=True)

    def scat_wait(ci, b):
        pltpu.make_async_copy(ones_v, acc_sh.at[idx_d.at[ci]],
                              ssems[b]).wait()

    for ci in range(4):
        scat_start(ci, ci)

    def step(t, carry):
        for k in range(4):
            ci = 4 + 4 * t + k
            scat_wait(ci - 4, k)
            scat_start(ci, k)
        return carry

    nloop4 = (NSTEPS - 4) // 4
    lax.fori_loop(0, nloop4, step, 0)
    for ci in range(4 + 4 * nloop4, NSTEPS):
        scat_wait(ci - 4, ci % 4)
        scat_start(ci, ci % 4)
    for ci in range(NSTEPS - 4, NSTEPS):
        scat_wait(ci, ci % 4)
    plsc.subcore_barrier()
    pltpu.sync_copy(acc_sh.at[pl.ds(s * ROWS_PT, ROWS_PT)],
                    out_hbm.at[c, pl.ds(s * ROWS_PT, ROWS_PT)])


_sc_degree = pl.kernel(
    _sc_degree_body,
    out_type=jax.ShapeDtypeStruct((NC, N_PAD, FEAT), jnp.float32),
    mesh=_SC_MESH,
    scratch_types=[
        pltpu.VMEM((NSTEPS, CHUNK), jnp.int32),
        pltpu.VMEM((CHUNK, FEAT), jnp.float32),
        pltpu.SemaphoreType.DMA,
        pltpu.SemaphoreType.DMA,
        pltpu.SemaphoreType.DMA,
        pltpu.SemaphoreType.DMA,
        pltpu.VMEM_SHARED((N_PAD, FEAT), jnp.float32),
    ],
)


def _sc_scatter_body(g_hbm, srcw_hbm, dstw_hbm, zeros_hbm, out_hbm,
                     idx_s, dst_v, rows_v,
                     gsem0, gsem1, dsem0, dsem1, ssem0, ssem1, acc_sh):
    c = lax.axis_index("c")
    s = lax.axis_index("s")
    wid = s * NC + c
    pltpu.sync_copy(zeros_hbm.at[pl.ds(s * ROWS_PT, ROWS_PT)],
                    acc_sh.at[pl.ds(s * ROWS_PT, ROWS_PT)])
    pltpu.sync_copy(srcw_hbm.at[wid], idx_s)
    plsc.subcore_barrier()
    gsems = (gsem0, gsem1)
    dsems = (dsem0, dsem1)
    ssems = (ssem0, ssem1)

    def gath_start(ci, b):
        pltpu.async_copy(g_hbm.at[idx_s.at[ci]], rows_v.at[b], gsems[b])

    def gath_wait(ci, b):
        pltpu.make_async_copy(g_hbm.at[idx_s.at[ci]], rows_v.at[b],
                              gsems[b]).wait()

    def dst_start(ci, b):
        pltpu.async_copy(dstw_hbm.at[wid, pl.ds(ci, 1)], dst_v.at[b],
                         dsems[b])

    def dst_wait(ci, b):
        pltpu.make_async_copy(dstw_hbm.at[wid, pl.ds(ci, 1)], dst_v.at[b],
                              dsems[b]).wait()

    def scat_start(ci, b):
        pltpu.async_copy(rows_v.at[b], acc_sh.at[dst_v.at[b, 0]], ssems[b],
                         add=True)

    def scat_wait(ci, b):
        pltpu.make_async_copy(rows_v.at[b], acc_sh.at[dst_v.at[b, 0]],
                              ssems[b]).wait()

    gath_start(0, 0)
    dst_start(0, 0)
    gath_start(1, 1)
    dst_start(1, 1)
    gath_wait(0, 0)
    dst_wait(0, 0)
    scat_start(0, 0)
    scat_wait(0, 0)
    gath_start(2, 0)
    dst_start(2, 0)
    gath_wait(1, 1)
    dst_wait(1, 1)
    scat_start(1, 1)

    def step(t, carry):
        for k in range(2):
            ci = 3 + 2 * t + k
            b = (1 + k) % 2
            ob = k % 2
            scat_wait(ci - 2, b)
            gath_start(ci, b)
            dst_start(ci, b)
            gath_wait(ci - 1, ob)
            dst_wait(ci - 1, ob)
            scat_start(ci - 1, ob)
        return carry

    lax.fori_loop(0, (NSTEPS - 3) // 2, step, 0)
    gath_wait(NSTEPS - 1, (NSTEPS - 1) % 2)
    dst_wait(NSTEPS - 1, (NSTEPS - 1) % 2)
    scat_start(NSTEPS - 1, (NSTEPS - 1) % 2)
    scat_wait(NSTEPS - 2, (NSTEPS - 2) % 2)
    scat_wait(NSTEPS - 1, (NSTEPS - 1) % 2)
    plsc.subcore_barrier()
    pltpu.sync_copy(acc_sh.at[pl.ds(s * ROWS_PT, ROWS_PT)],
                    out_hbm.at[c, pl.ds(s * ROWS_PT, ROWS_PT)])


_sc_scatter = pl.kernel(
    _sc_scatter_body,
    out_type=jax.ShapeDtypeStruct((NC, N_PAD, FEAT), jnp.float32),
    mesh=_SC_MESH,
    scratch_types=[
        pltpu.VMEM((NSTEPS, CHUNK), jnp.int32),
        pltpu.VMEM((2, 1, CHUNK), jnp.int32),
        pltpu.VMEM((2, CHUNK, FEAT), jnp.float32),
        pltpu.SemaphoreType.DMA,
        pltpu.SemaphoreType.DMA,
        pltpu.SemaphoreType.DMA,
        pltpu.SemaphoreType.DMA,
        pltpu.SemaphoreType.DMA,
        pltpu.SemaphoreType.DMA,
        pltpu.VMEM_SHARED((N_PAD, FEAT), jnp.float32),
    ],
)


def _tc_enc_body(x_ref, encW_ref, encb_ref, W1_ref, m1_ref):
    h0 = jnp.dot(x_ref[...], encW_ref[...],
                 preferred_element_type=jnp.float32) + encb_ref[...]
    m1_ref[...] = jnp.dot(h0, W1_ref[...], preferred_element_type=jnp.float32)


def _tc_enc(x, enc_W, enc_b2, W1):
    return pl.pallas_call(
        _tc_enc_body,
        grid=(NBLK,),
        in_specs=[
            pl.BlockSpec((ROWBLK, FEAT), lambda i: (i, 0)),
            pl.BlockSpec((FEAT, FEAT), lambda i: (0, 0)),
            pl.BlockSpec((1, FEAT), lambda i: (0, 0)),
            pl.BlockSpec((FEAT, FEAT), lambda i: (0, 0)),
        ],
        out_specs=pl.BlockSpec((ROWBLK, FEAT), lambda i: (i, 0)),
        out_shape=jax.ShapeDtypeStruct((N_NODES, FEAT), jnp.float32),
    )(x, enc_W, enc_b2, W1)


def _tc_g1_body(degp_ref, m1_ref, g1_ref, dinv_ref):
    deg = degp_ref[0, :, 0:1] + degp_ref[1, :, 0:1] + 1.0
    dinv = lax.rsqrt(deg)
    g1_ref[...] = dinv * m1_ref[...]
    dinv_ref[...] = jnp.broadcast_to(dinv, (ROWBLK, 8))


def _tc_g1(deg_parts, m1):
    return pl.pallas_call(
        _tc_g1_body,
        grid=(NBLK,),
        in_specs=[
            pl.BlockSpec((NC, ROWBLK, FEAT), lambda i: (0, i, 0)),
            pl.BlockSpec((ROWBLK, FEAT), lambda i: (i, 0)),
        ],
        out_specs=[
            pl.BlockSpec((ROWBLK, FEAT), lambda i: (i, 0)),
            pl.BlockSpec((ROWBLK, 8), lambda i: (i, 0)),
        ],
        out_shape=[
            jax.ShapeDtypeStruct((N_NODES, FEAT), jnp.float32),
            jax.ShapeDtypeStruct((N_NODES, 8), jnp.float32),
        ],
    )(deg_parts, m1)


def _tc_mid_body(acc_ref, g_ref, dinv_ref, b_ref, W_ref, out_ref):
    dinv = dinv_ref[:, 0:1]
    h = jnp.maximum(
        dinv * (acc_ref[0] + acc_ref[1] + g_ref[...]) + b_ref[...], 0.0)
    out_ref[...] = dinv * jnp.dot(h, W_ref[...],
                                  preferred_element_type=jnp.float32)


def _tc_mid(acc, g, dinv8, b2d, W):
    return pl.pallas_call(
        _tc_mid_body,
        grid=(NBLK,),
        in_specs=[
            pl.BlockSpec((NC, ROWBLK, FEAT), lambda i: (0, i, 0)),
            pl.BlockSpec((ROWBLK, FEAT), lambda i: (i, 0)),
            pl.BlockSpec((ROWBLK, 8), lambda i: (i, 0)),
            pl.BlockSpec((1, FEAT), lambda i: (0, 0)),
            pl.BlockSpec((FEAT, FEAT), lambda i: (0, 0)),
        ],
        out_specs=pl.BlockSpec((ROWBLK, FEAT), lambda i: (i, 0)),
        out_shape=jax.ShapeDtypeStruct((N_NODES, FEAT), jnp.float32),
    )(acc, g, dinv8, b2d, W)


def _tc_final_body(acc_ref, g_ref, dinv_ref, b_ref, batch_ref, outW_ref,
                   outb_ref, out_ref, sums_ref, counts_ref):
    i = pl.program_id(0)
    dinv = dinv_ref[:, 0:1]
    h = jnp.maximum(
        dinv * (acc_ref[0] + acc_ref[1] + g_ref[...]) + b_ref[...], 0.0)
    onehot = (batch_ref[:, 0:1] ==
              lax.broadcasted_iota(jnp.int32, (ROWBLK, N_GRAPHS), 1)
              ).astype(jnp.float32)
    psum = lax.dot_general(onehot, h, (((0,), (0,)), ((), ())),
                           preferred_element_type=jnp.float32)
    pcnt = lax.dot_general(onehot, jnp.ones_like(h), (((0,), (0,)), ((), ())),
                           preferred_element_type=jnp.float32)

    @pl.when(i == 0)
    def _():
        sums_ref[...] = jnp.zeros_like(sums_ref)
        counts_ref[...] = jnp.zeros_like(counts_ref)

    sums_ref[...] += psum
    counts_ref[...] += pcnt

    @pl.when(i == pl.num_programs(0) - 1)
    def _():
        pooled = sums_ref[...] / jnp.maximum(counts_ref[...], 1.0)
        out_ref[...] = jnp.dot(pooled, outW_ref[...],
                               preferred_element_type=jnp.float32) + outb_ref[...]


def _tc_final(acc, g, dinv8, b2d, batch8, out_W, out_b2):
    return pl.pallas_call(
        _tc_final_body,
        grid=(NBLK,),
        in_specs=[
            pl.BlockSpec((NC, ROWBLK, FEAT), lambda i: (0, i, 0)),
            pl.BlockSpec((ROWBLK, FEAT), lambda i: (i, 0)),
            pl.BlockSpec((ROWBLK, 8), lambda i: (i, 0)),
            pl.BlockSpec((1, FEAT), lambda i: (0, 0)),
            pl.BlockSpec((ROWBLK, 8), lambda i: (i, 0)),
            pl.BlockSpec((FEAT, N_CLASS), lambda i: (0, 0)),
            pl.BlockSpec((1, N_CLASS), lambda i: (0, 0)),
        ],
        out_specs=pl.BlockSpec((N_GRAPHS, N_CLASS), lambda i: (0, 0)),
        out_shape=jax.ShapeDtypeStruct((N_GRAPHS, N_CLASS), jnp.float32),
        scratch_shapes=[
            pltpu.VMEM((N_GRAPHS, FEAT), jnp.float32),
            pltpu.VMEM((N_GRAPHS, FEAT), jnp.float32),
        ],
    )(acc, g, dinv8, b2d, batch8, out_W, out_b2)


def kernel(x, edge_index, batch, enc_W, enc_b, W1, b1, W2, b2, out_W, out_b):
    src = edge_index[0].astype(jnp.int32)
    dst = edge_index[1].astype(jnp.int32)
    pad_e = E_PAD - N_EDGES
    src_pad = jnp.arange(pad_e, dtype=jnp.int32) % N_NODES
    dst_pad = N_NODES + jnp.arange(pad_e, dtype=jnp.int32) % (N_PAD - N_NODES)
    srcw = jnp.concatenate([src, src_pad]).reshape(NW, NSTEPS, CHUNK)
    dstw = jnp.concatenate([dst, dst_pad]).reshape(NW, NSTEPS, CHUNK)
    zeros128 = jnp.zeros((N_PAD, FEAT), jnp.float32)
    batch8 = jnp.broadcast_to(batch.astype(jnp.int32)[:, None], (N_NODES, 8))
    enc_b2 = enc_b.reshape(1, FEAT)
    b1_2 = b1.reshape(1, FEAT)
    b2_2 = b2.reshape(1, FEAT)
    out_b2 = out_b.reshape(1, N_CLASS)

    ones128 = jnp.ones((CHUNK, FEAT), jnp.float32)
    deg_parts = _sc_degree(dstw, zeros128, ones128)
    m1 = _tc_enc(x, enc_W, enc_b2, W1)
    g1, dinv8 = _tc_g1(deg_parts, m1)
    acc1 = _sc_scatter(g1, srcw, dstw, zeros128)
    g2 = _tc_mid(acc1, g1, dinv8, b1_2, W2)
    acc2 = _sc_scatter(g2, srcw, dstw, zeros128)
    return _tc_final(acc2, g2, dinv8, b2_2, batch8, out_W, out_b2)
